# baseline (speedup 1.0000x reference)
import functools

import numpy as np

import jax
import jax.numpy as jnp
from jax import lax
from jax.experimental import pallas as pl
from jax.experimental.pallas import tpu as pltpu
from jax.experimental.pallas import tpu_sc as plsc

N = 10000
NE = 320000
D = 128
NC, NS, L = 2, 16, 16
NW = NC * NS
G = 64
K = 160
CH = 8
NBLK = K // CH
EPAD = NW * K * G
NPAD = 10240
ROWS_PER_TILE = NPAD // NS
DEG_PER_TILE = NPAD // NS


def _lin1_body(x_ref, w_ref, b_ref, o_ref):
    o_ref[...] = (
        jnp.dot(x_ref[...], w_ref[...], preferred_element_type=jnp.float32)
        + b_ref[...]
    )


def _lin1(x, W1, b1r):
    grid = (10,)
    blk = N // grid[0]
    return pl.pallas_call(
        _lin1_body,
        grid=grid,
        in_specs=[
            pl.BlockSpec((blk, D), lambda i: (i, 0)),
            pl.BlockSpec((D, D), lambda i: (0, 0)),
            pl.BlockSpec((1, D), lambda i: (0, 0)),
        ],
        out_specs=pl.BlockSpec((blk, D), lambda i: (i, 0)),
        out_shape=jax.ShapeDtypeStruct((N, D), jnp.float32),
    )(x, W1, b1r)


def _combine_body(x_ref, s0_ref, s1_ref, d0_ref, d1_ref, w2_ref, w3_ref,
                  b3_ref, o_ref):
    x = x_ref[...]
    b = jnp.dot(x, w2_ref[...], preferred_element_type=jnp.float32)
    c = jnp.dot(x, w3_ref[...], preferred_element_type=jnp.float32) + b3_ref[...]
    d = d0_ref[...] + d1_ref[...]
    o_ref[...] = jnp.maximum(s0_ref[0] + s1_ref[0] - d * b + c, 0.0)


def _combine(x, S, d0, d1, W2, W3, b3r):
    grid = (10,)
    blk = N // grid[0]
    row_spec = pl.BlockSpec((blk, D), lambda i: (i, 0))
    return pl.pallas_call(
        _combine_body,
        grid=grid,
        in_specs=[
            row_spec,
            pl.BlockSpec((1, blk, D), lambda i: (0, i, 0)),
            pl.BlockSpec((1, blk, D), lambda i: (1, i, 0)),
            pl.BlockSpec((blk, 1), lambda i: (i, 0)),
            pl.BlockSpec((blk, 1), lambda i: (i, 0)),
            pl.BlockSpec((D, D), lambda i: (0, 0)),
            pl.BlockSpec((D, D), lambda i: (0, 0)),
            pl.BlockSpec((1, D), lambda i: (0, 0)),
        ],
        out_specs=row_spec,
        out_shape=jax.ShapeDtypeStruct((N, D), jnp.float32),
    )(x, S, S, d0, d1, W2, W3, b3r)


def _edge_scatter_body(a_hbm, src_hbm, dst_hbm, w_hbm, s_out, deg_out,
                       src_r, dst_r, w_r, gbuf0, gbuf1, sbuf0, sbuf1, zbuf,
                       S_sh, deg_sh, gsem0, gsem1, ssem0, ssem1):
    c = lax.axis_index("c")
    s = lax.axis_index("s")
    wid = c * NS + s

    zeros16 = jnp.zeros((L,), jnp.float32)

    def zero_rows(i, _):
        for j in range(D // L):
            gbuf0[i, pl.ds(j * L, L)] = zeros16
        return 0
    lax.fori_loop(0, G, zero_rows, 0)

    def zero_zbuf(i, _):
        zbuf[pl.ds(pl.multiple_of(i * L, L), L)] = zeros16
        return 0
    lax.fori_loop(0, DEG_PER_TILE // L, zero_zbuf, 0)

    for kk in range(ROWS_PER_TILE // G):
        pltpu.sync_copy(gbuf0,
                        S_sh.at[pl.ds(s * ROWS_PER_TILE + kk * G, G)])
    pltpu.sync_copy(zbuf.at[pl.ds(0, DEG_PER_TILE)],
                    deg_sh.at[pl.ds(s * DEG_PER_TILE, DEG_PER_TILE)])
    plsc.subcore_barrier()

    gbufs = (gbuf0, gbuf1)
    sbufs = (sbuf0, sbuf1)
    gsems = (gsem0, gsem1)
    ssems = (ssem0, ssem1)

    def ring_row(g):
        return ((g // CH) % 2) * CH + (g % CH)

    def stage_block(i):
        sl_hbm = pl.ds(i * CH, CH)
        sl_ring = pl.ds((i % 2) * CH, CH)
        pltpu.sync_copy(src_hbm.at[wid, sl_hbm], src_r.at[sl_ring])
        pltpu.sync_copy(dst_hbm.at[wid, sl_hbm], dst_r.at[sl_ring])
        pltpu.sync_copy(w_hbm.at[wid, sl_hbm], w_r.at[sl_ring])

    def issue_gather(g, b):
        pltpu.async_copy(a_hbm.at[src_r.at[ring_row(g)]], gbufs[b], gsems[b])

    def wait_gather(g, b):
        pltpu.make_async_copy(a_hbm.at[src_r.at[ring_row(g)]], gbufs[b],
                              gsems[b]).wait()

    def issue_scatter(g, b):
        rr = ring_row(g)
        pltpu.async_copy(sbufs[b], S_sh.at[dst_r.at[rr]], ssems[b], add=True)
        pltpu.async_copy(w_r.at[rr], deg_sh.at[dst_r.at[rr]], ssems[b], add=True)

    def wait_scatter(g, b):
        rr = ring_row(g)
        pltpu.make_async_copy(sbufs[b], S_sh.at[dst_r.at[rr]], ssems[b]).wait()
        pltpu.make_async_copy(w_r.at[rr], deg_sh.at[dst_r.at[rr]], ssems[b]).wait()

    def mul_scale(g, b):
        gbuf = gbufs[b]
        sbuf = sbufs[b]
        rr = ring_row(g)

        def mul_rows(t, _):
            wv = w_r[rr, pl.ds(pl.multiple_of(t * L, L), L)]
            for l in range(L):
                wi = wv[l]
                i = t * L + l
                for j in range(D // L):
                    sl = pl.ds(j * L, L)
                    sbuf[i, sl] = gbuf[i, sl] * wi
            return 0
        lax.fori_loop(0, G // L, mul_rows, 0)

    stage_block(0)
    issue_gather(0, 0)
    issue_gather(1, 1)

    def block(i, _):
        def step(g, b):
            wait_gather(g, b)

            @pl.when(g >= 2)
            def _():
                wait_scatter(g - 2, b)

            mul_scale(g, b)

            @pl.when(g + 2 < K)
            def _():
                issue_gather(g + 2, b)

            issue_scatter(g, b)

        def pair(k):
            g0 = i * CH + 2 * k
            step(g0, 0)
            step(g0 + 1, 1)

        pair(0)

        @pl.when(i + 1 < NBLK)
        def _():
            stage_block(i + 1)

        def pair_body(k, _):
            pair(k)
            return 0
        lax.fori_loop(1, CH // 2, pair_body, 0)
        return 0

    lax.fori_loop(0, NBLK, block, 0)
    wait_scatter(K - 2, 0)
    wait_scatter(K - 1, 1)
    plsc.subcore_barrier()

    for kk in range(ROWS_PER_TILE // G):
        r0 = s * ROWS_PER_TILE + kk * G
        pltpu.sync_copy(S_sh.at[pl.ds(r0, G)], s_out.at[c, pl.ds(r0, G)])
    pltpu.sync_copy(deg_sh.at[pl.ds(s * DEG_PER_TILE, DEG_PER_TILE)],
                    deg_out.at[c, pl.ds(s * DEG_PER_TILE, DEG_PER_TILE)])


def _edge_scatter(a, src_p, dst_p, w_p):
    mesh = plsc.VectorSubcoreMesh(core_axis_name="c", subcore_axis_name="s")
    return pl.kernel(
        _edge_scatter_body,
        out_type=[
            jax.ShapeDtypeStruct((NC, NPAD, D), jnp.float32),
            jax.ShapeDtypeStruct((NC, NPAD), jnp.float32),
        ],
        mesh=mesh,
        scratch_types=[
            pltpu.VMEM((2 * CH, G), jnp.int32),
            pltpu.VMEM((2 * CH, G), jnp.int32),
            pltpu.VMEM((2 * CH, G), jnp.float32),
            pltpu.VMEM((G, D), jnp.float32),
            pltpu.VMEM((G, D), jnp.float32),
            pltpu.VMEM((G, D), jnp.float32),
            pltpu.VMEM((G, D), jnp.float32),
            pltpu.VMEM((DEG_PER_TILE,), jnp.float32),
            pltpu.VMEM_SHARED((NPAD, D), jnp.float32),
            pltpu.VMEM_SHARED((NPAD,), jnp.float32),
            pltpu.SemaphoreType.DMA,
            pltpu.SemaphoreType.DMA,
            pltpu.SemaphoreType.DMA,
            pltpu.SemaphoreType.DMA,
        ],
    )(a, src_p, dst_p, w_p)


def kernel(x, E_idx, E_w, W1, b1, W2, W3, b3):
    src = E_idx[0].astype(jnp.int32)
    dst = E_idx[1].astype(jnp.int32)
    npad = EPAD - NE
    pad_idx = jnp.asarray(np.arange(npad, dtype=np.int32) % N)
    src_p = jnp.concatenate([src, pad_idx]).reshape(NW, K, G)
    dst_p = jnp.concatenate([dst, pad_idx]).reshape(NW, K, G)
    w_p = jnp.concatenate(
        [E_w, jnp.asarray(np.zeros(npad, np.float32))]).reshape(NW, K, G)

    b1r = b1.reshape(1, D)
    b3r = b3.reshape(1, D)

    a = _lin1(x, W1, b1r)
    S, deg = _edge_scatter(a, src_p, dst_p, w_p)
    d0 = deg[0, :N].reshape(N, 1)
    d1 = deg[1, :N].reshape(N, 1)
    return _combine(x, S, d0, d1, W2, W3, b3r)


# speedup vs baseline: 1.1545x; 1.1545x over previous
import functools

import numpy as np

import jax
import jax.numpy as jnp
from jax import lax
from jax.experimental import pallas as pl
from jax.experimental.pallas import tpu as pltpu
from jax.experimental.pallas import tpu_sc as plsc

N = 10000
NE = 320000
D = 128
NC, NS, L = 2, 16, 16
NW = NC * NS
G = 128
K = 80
CH = 16
NBLK = K // CH
EPAD = NW * K * G
NPAD = 10240
ROWS_PER_TILE = NPAD // NS
DEG_PER_TILE = NPAD // NS


def _lin1_body(x_ref, w_ref, b_ref, o_ref):
    o_ref[...] = (
        jnp.dot(x_ref[...], w_ref[...], preferred_element_type=jnp.float32)
        + b_ref[...]
    )


def _lin1(x, W1, b1r):
    grid = (10,)
    blk = N // grid[0]
    return pl.pallas_call(
        _lin1_body,
        grid=grid,
        in_specs=[
            pl.BlockSpec((blk, D), lambda i: (i, 0)),
            pl.BlockSpec((D, D), lambda i: (0, 0)),
            pl.BlockSpec((1, D), lambda i: (0, 0)),
        ],
        out_specs=pl.BlockSpec((blk, D), lambda i: (i, 0)),
        out_shape=jax.ShapeDtypeStruct((N, D), jnp.float32),
    )(x, W1, b1r)


def _combine_body(x_ref, s_ref, d0_ref, d1_ref, w2_ref, w3_ref,
                  b3_ref, o_ref):
    x = x_ref[...]
    b = jnp.dot(x, w2_ref[...], preferred_element_type=jnp.float32)
    c = jnp.dot(x, w3_ref[...], preferred_element_type=jnp.float32) + b3_ref[...]
    d = d0_ref[...] + d1_ref[...]
    o_ref[...] = jnp.maximum(s_ref[0] + s_ref[1] - d * b + c, 0.0)


def _combine(x, S, d0, d1, W2, W3, b3r):
    grid = (10,)
    blk = N // grid[0]
    row_spec = pl.BlockSpec((blk, D), lambda i: (i, 0))
    return pl.pallas_call(
        _combine_body,
        grid=grid,
        in_specs=[
            row_spec,
            pl.BlockSpec((NC, blk, D), lambda i: (0, i, 0)),
            pl.BlockSpec((blk, 1), lambda i: (i, 0)),
            pl.BlockSpec((blk, 1), lambda i: (i, 0)),
            pl.BlockSpec((D, D), lambda i: (0, 0)),
            pl.BlockSpec((D, D), lambda i: (0, 0)),
            pl.BlockSpec((1, D), lambda i: (0, 0)),
        ],
        out_specs=row_spec,
        out_shape=jax.ShapeDtypeStruct((N, D), jnp.float32),
    )(x, S, d0, d1, W2, W3, b3r)


def _edge_scatter_body(a_hbm, src_hbm, dst_hbm, w_hbm, s_out, deg_out,
                       src_r, dst_r, w_r, gbuf0, gbuf1, zbuf,
                       S_sh, deg_sh, gsem0, gsem1, ssem0, ssem1):
    c = lax.axis_index("c")
    s = lax.axis_index("s")
    wid = c * NS + s

    zeros16 = jnp.zeros((L,), jnp.float32)

    def zero_rows(i, _):
        for j in range(D // L):
            gbuf0[i, pl.ds(j * L, L)] = zeros16
        return 0
    lax.fori_loop(0, G, zero_rows, 0)

    def zero_zbuf(i, _):
        zbuf[pl.ds(pl.multiple_of(i * L, L), L)] = zeros16
        return 0
    lax.fori_loop(0, DEG_PER_TILE // L, zero_zbuf, 0)

    for kk in range(ROWS_PER_TILE // G):
        pltpu.sync_copy(gbuf0,
                        S_sh.at[pl.ds(s * ROWS_PER_TILE + kk * G, G)])
    pltpu.sync_copy(zbuf.at[pl.ds(0, DEG_PER_TILE)],
                    deg_sh.at[pl.ds(s * DEG_PER_TILE, DEG_PER_TILE)])
    plsc.subcore_barrier()

    gbufs = (gbuf0, gbuf1)
    gsems = (gsem0, gsem1)
    ssems = (ssem0, ssem1)

    def ring_row(g):
        return ((g // CH) % 2) * CH + (g % CH)

    def stage_block(i):
        sl_hbm = pl.ds(i * CH, CH)
        sl_ring = pl.ds((i % 2) * CH, CH)
        pltpu.sync_copy(src_hbm.at[wid, sl_hbm], src_r.at[sl_ring])
        pltpu.sync_copy(dst_hbm.at[wid, sl_hbm], dst_r.at[sl_ring])
        pltpu.sync_copy(w_hbm.at[wid, sl_hbm], w_r.at[sl_ring])

    def issue_gather(g, b):
        pltpu.async_copy(a_hbm.at[src_r.at[ring_row(g)]], gbufs[b], gsems[b])

    def wait_gather(g, b):
        pltpu.make_async_copy(a_hbm.at[src_r.at[ring_row(g)]], gbufs[b],
                              gsems[b]).wait()

    def issue_scatter(g, b):
        rr = ring_row(g)
        pltpu.async_copy(gbufs[b], S_sh.at[dst_r.at[rr]], ssems[b], add=True)
        pltpu.async_copy(w_r.at[rr], deg_sh.at[dst_r.at[rr]], ssems[b], add=True)

    def wait_scatter(g, b):
        rr = ring_row(g)
        pltpu.make_async_copy(gbufs[b], S_sh.at[dst_r.at[rr]], ssems[b]).wait()
        pltpu.make_async_copy(w_r.at[rr], deg_sh.at[dst_r.at[rr]], ssems[b]).wait()

    def mul_in_place(g, b):
        gbuf = gbufs[b]
        rr = ring_row(g)

        def mul_rows(t, _):
            wv = w_r[rr, pl.ds(pl.multiple_of(t * L, L), L)]
            for l in range(L):
                wi = wv[l]
                i = t * L + l
                for j in range(D // L):
                    sl = pl.ds(j * L, L)
                    gbuf[i, sl] = gbuf[i, sl] * wi
            return 0
        lax.fori_loop(0, G // L, mul_rows, 0)

    stage_block(0)
    issue_gather(0, 0)

    def block(i, _):
        def step(g, b):
            wait_gather(g, b)
            mul_in_place(g, b)
            issue_scatter(g, b)

            @pl.when(g >= 1)
            def _():
                wait_scatter(g - 1, 1 - b)

            @pl.when(g + 1 < K)
            def _():
                issue_gather(g + 1, 1 - b)

        def pair(k):
            g0 = i * CH + 2 * k
            step(g0, 0)
            step(g0 + 1, 1)

        pair(0)

        @pl.when(i + 1 < NBLK)
        def _():
            stage_block(i + 1)

        def pair_body(k, _):
            pair(k)
            return 0
        lax.fori_loop(1, CH // 2, pair_body, 0)
        return 0

    lax.fori_loop(0, NBLK, block, 0)
    wait_scatter(K - 1, 1)
    plsc.subcore_barrier()

    for kk in range(ROWS_PER_TILE // G):
        r0 = s * ROWS_PER_TILE + kk * G
        pltpu.sync_copy(S_sh.at[pl.ds(r0, G)], s_out.at[c, pl.ds(r0, G)])
    pltpu.sync_copy(deg_sh.at[pl.ds(s * DEG_PER_TILE, DEG_PER_TILE)],
                    deg_out.at[c, pl.ds(s * DEG_PER_TILE, DEG_PER_TILE)])


def _edge_scatter(a, src_p, dst_p, w_p):
    mesh = plsc.VectorSubcoreMesh(core_axis_name="c", subcore_axis_name="s")
    return pl.kernel(
        _edge_scatter_body,
        out_type=[
            jax.ShapeDtypeStruct((NC, NPAD, D), jnp.float32),
            jax.ShapeDtypeStruct((NC, NPAD), jnp.float32),
        ],
        mesh=mesh,
        scratch_types=[
            pltpu.VMEM((2 * CH, G), jnp.int32),
            pltpu.VMEM((2 * CH, G), jnp.int32),
            pltpu.VMEM((2 * CH, G), jnp.float32),
            pltpu.VMEM((G, D), jnp.float32),
            pltpu.VMEM((G, D), jnp.float32),
            pltpu.VMEM((DEG_PER_TILE,), jnp.float32),
            pltpu.VMEM_SHARED((NPAD, D), jnp.float32),
            pltpu.VMEM_SHARED((NPAD,), jnp.float32),
            pltpu.SemaphoreType.DMA,
            pltpu.SemaphoreType.DMA,
            pltpu.SemaphoreType.DMA,
            pltpu.SemaphoreType.DMA,
        ],
    )(a, src_p, dst_p, w_p)


def kernel(x, E_idx, E_w, W1, b1, W2, W3, b3):
    src = E_idx[0].astype(jnp.int32)
    dst = E_idx[1].astype(jnp.int32)
    npad = EPAD - NE
    pad_idx = jnp.asarray(np.arange(npad, dtype=np.int32) % N)
    src_p = jnp.concatenate([src, pad_idx]).reshape(NW, K, G)
    dst_p = jnp.concatenate([dst, pad_idx]).reshape(NW, K, G)
    w_p = jnp.concatenate(
        [E_w, jnp.asarray(np.zeros(npad, np.float32))]).reshape(NW, K, G)

    b1r = b1.reshape(1, D)
    b3r = b3.reshape(1, D)

    a = _lin1(x, W1, b1r)
    S, deg = _edge_scatter(a, src_p, dst_p, w_p)
    d0 = deg[0, :N].reshape(N, 1)
    d1 = deg[1, :N].reshape(N, 1)
    return _combine(x, S, d0, d1, W2, W3, b3r)


# speedup vs baseline: 1.2058x; 1.0444x over previous
import functools

import numpy as np

import jax
import jax.numpy as jnp
from jax import lax
from jax.experimental import pallas as pl
from jax.experimental.pallas import tpu as pltpu
from jax.experimental.pallas import tpu_sc as plsc

N = 10000
NE = 320000
D = 128
NC, NS, L = 2, 16, 16
NW = NC * NS
G = 128
K = 80
CH = 16
NBLK = K // CH
NGRP = NE // G
NGRP_V = NW * K
TBLK0 = 156
TSTART = TBLK0 * CH
TAIL = NGRP_V - TSTART
NPAD = 10240
ROWS_PER_TILE = NPAD // NS
DEG_PER_TILE = NPAD // NS


def _lin1_body(x_ref, w_ref, b_ref, o_ref):
    o_ref[...] = (
        jnp.dot(x_ref[...], w_ref[...], preferred_element_type=jnp.float32)
        + b_ref[...]
    )


def _lin1(x, W1, b1r):
    grid = (10,)
    blk = N // grid[0]
    return pl.pallas_call(
        _lin1_body,
        grid=grid,
        in_specs=[
            pl.BlockSpec((blk, D), lambda i: (i, 0)),
            pl.BlockSpec((D, D), lambda i: (0, 0)),
            pl.BlockSpec((1, D), lambda i: (0, 0)),
        ],
        out_specs=pl.BlockSpec((blk, D), lambda i: (i, 0)),
        out_shape=jax.ShapeDtypeStruct((N, D), jnp.float32),
    )(x, W1, b1r)


def _combine_body(x_ref, s_ref, d0_ref, d1_ref, w2_ref, w3_ref,
                  b3_ref, o_ref):
    x = x_ref[...]
    b = jnp.dot(x, w2_ref[...], preferred_element_type=jnp.float32)
    c = jnp.dot(x, w3_ref[...], preferred_element_type=jnp.float32) + b3_ref[...]
    d = d0_ref[...] + d1_ref[...]
    o_ref[...] = jnp.maximum(s_ref[0] + s_ref[1] - d * b + c, 0.0)


def _combine(x, S, d0, d1, W2, W3, b3r):
    grid = (10,)
    blk = N // grid[0]
    row_spec = pl.BlockSpec((blk, D), lambda i: (i, 0))
    return pl.pallas_call(
        _combine_body,
        grid=grid,
        in_specs=[
            row_spec,
            pl.BlockSpec((NC, blk, D), lambda i: (0, i, 0)),
            pl.BlockSpec((blk, 1), lambda i: (i, 0)),
            pl.BlockSpec((blk, 1), lambda i: (i, 0)),
            pl.BlockSpec((D, D), lambda i: (0, 0)),
            pl.BlockSpec((D, D), lambda i: (0, 0)),
            pl.BlockSpec((1, D), lambda i: (0, 0)),
        ],
        out_specs=row_spec,
        out_shape=jax.ShapeDtypeStruct((N, D), jnp.float32),
    )(x, S, d0, d1, W2, W3, b3r)


def _edge_scatter_body(a_hbm, e3_hbm, ew_hbm, tidx_hbm, tw_hbm, s_out, deg_out,
                       src_r, dst_r, w_r, gbuf0, gbuf1, zbuf,
                       S_sh, deg_sh, gsem0, gsem1, ssem0, ssem1):
    c = lax.axis_index("c")
    s = lax.axis_index("s")
    wid = c * NS + s

    zeros16 = jnp.zeros((L,), jnp.float32)

    def zero_rows(i, _):
        for j in range(D // L):
            gbuf0[i, pl.ds(j * L, L)] = zeros16
        return 0
    lax.fori_loop(0, G, zero_rows, 0)

    def zero_zbuf(i, _):
        zbuf[pl.ds(pl.multiple_of(i * L, L), L)] = zeros16
        return 0
    lax.fori_loop(0, DEG_PER_TILE // L, zero_zbuf, 0)

    for kk in range(ROWS_PER_TILE // G):
        pltpu.sync_copy(gbuf0,
                        S_sh.at[pl.ds(s * ROWS_PER_TILE + kk * G, G)])
    pltpu.sync_copy(zbuf.at[pl.ds(0, DEG_PER_TILE)],
                    deg_sh.at[pl.ds(s * DEG_PER_TILE, DEG_PER_TILE)])
    plsc.subcore_barrier()

    gbufs = (gbuf0, gbuf1)
    gsems = (gsem0, gsem1)
    ssems = (ssem0, ssem1)

    def ring_row(g):
        return ((g // CH) % 2) * CH + (g % CH)

    def stage_block(i):
        B = wid * NBLK + i
        sl_ring = pl.ds((i % 2) * CH, CH)

        @pl.when(B < TBLK0)
        def _():
            sl = pl.ds(pl.multiple_of(B * CH, CH), CH)
            pltpu.sync_copy(e3_hbm.at[0, sl], src_r.at[sl_ring])
            pltpu.sync_copy(e3_hbm.at[1, sl], dst_r.at[sl_ring])
            pltpu.sync_copy(ew_hbm.at[sl], w_r.at[sl_ring])

        @pl.when(B >= TBLK0)
        def _():
            sl = pl.ds(pl.multiple_of((B - TBLK0) * CH, CH), CH)
            pltpu.sync_copy(tidx_hbm.at[0, sl], src_r.at[sl_ring])
            pltpu.sync_copy(tidx_hbm.at[1, sl], dst_r.at[sl_ring])
            pltpu.sync_copy(tw_hbm.at[sl], w_r.at[sl_ring])

    def issue_gather(g, b):
        pltpu.async_copy(a_hbm.at[src_r.at[ring_row(g)]], gbufs[b], gsems[b])

    def wait_gather(g, b):
        pltpu.make_async_copy(a_hbm.at[src_r.at[ring_row(g)]], gbufs[b],
                              gsems[b]).wait()

    def issue_scatter(g, b):
        rr = ring_row(g)
        pltpu.async_copy(gbufs[b], S_sh.at[dst_r.at[rr]], ssems[b], add=True)
        pltpu.async_copy(w_r.at[rr], deg_sh.at[dst_r.at[rr]], ssems[b], add=True)

    def wait_scatter(g, b):
        rr = ring_row(g)
        pltpu.make_async_copy(gbufs[b], S_sh.at[dst_r.at[rr]], ssems[b]).wait()
        pltpu.make_async_copy(w_r.at[rr], deg_sh.at[dst_r.at[rr]], ssems[b]).wait()

    def mul_in_place(g, b):
        gbuf = gbufs[b]
        rr = ring_row(g)

        def mul_rows(t, _):
            wv = w_r[rr, pl.ds(pl.multiple_of(t * L, L), L)]
            for l in range(L):
                wi = wv[l]
                i = t * L + l
                for j in range(D // L):
                    sl = pl.ds(j * L, L)
                    gbuf[i, sl] = gbuf[i, sl] * wi
            return 0
        lax.fori_loop(0, G // L, mul_rows, 0)

    stage_block(0)
    issue_gather(0, 0)

    def block(i, _):
        def step(g, b):
            wait_gather(g, b)
            mul_in_place(g, b)
            issue_scatter(g, b)

            @pl.when(g >= 1)
            def _():
                wait_scatter(g - 1, 1 - b)

            @pl.when(g + 1 < K)
            def _():
                issue_gather(g + 1, 1 - b)

        def pair(k):
            g0 = i * CH + 2 * k
            step(g0, 0)
            step(g0 + 1, 1)

        pair(0)

        @pl.when(i + 1 < NBLK)
        def _():
            stage_block(i + 1)

        def pair_body(k, _):
            pair(k)
            return 0
        lax.fori_loop(1, CH // 2, pair_body, 0)
        return 0

    lax.fori_loop(0, NBLK, block, 0)
    wait_scatter(K - 1, 1)
    plsc.subcore_barrier()

    for kk in range(ROWS_PER_TILE // G):
        r0 = s * ROWS_PER_TILE + kk * G
        pltpu.sync_copy(S_sh.at[pl.ds(r0, G)], s_out.at[c, pl.ds(r0, G)])
    pltpu.sync_copy(deg_sh.at[pl.ds(s * DEG_PER_TILE, DEG_PER_TILE)],
                    deg_out.at[c, pl.ds(s * DEG_PER_TILE, DEG_PER_TILE)])


def _edge_scatter(a, E3, Ew2, tail_idx, tail_w):
    mesh = plsc.VectorSubcoreMesh(core_axis_name="c", subcore_axis_name="s")
    return pl.kernel(
        _edge_scatter_body,
        out_type=[
            jax.ShapeDtypeStruct((NC, NPAD, D), jnp.float32),
            jax.ShapeDtypeStruct((NC, NPAD), jnp.float32),
        ],
        mesh=mesh,
        scratch_types=[
            pltpu.VMEM((2 * CH, G), jnp.int32),
            pltpu.VMEM((2 * CH, G), jnp.int32),
            pltpu.VMEM((2 * CH, G), jnp.float32),
            pltpu.VMEM((G, D), jnp.float32),
            pltpu.VMEM((G, D), jnp.float32),
            pltpu.VMEM((DEG_PER_TILE,), jnp.float32),
            pltpu.VMEM_SHARED((NPAD, D), jnp.float32),
            pltpu.VMEM_SHARED((NPAD,), jnp.float32),
            pltpu.SemaphoreType.DMA,
            pltpu.SemaphoreType.DMA,
            pltpu.SemaphoreType.DMA,
            pltpu.SemaphoreType.DMA,
        ],
    )(a, E3, Ew2, tail_idx, tail_w)


def kernel(x, E_idx, E_w, W1, b1, W2, W3, b3):
    E3 = E_idx.astype(jnp.int32).reshape(2, NGRP, G)
    Ew2 = E_w.reshape(NGRP, G)
    npadg = NGRP_V - NGRP
    pad3 = jnp.asarray(
        (np.arange(2 * npadg * G, dtype=np.int32) % N).reshape(2, npadg, G))
    tail_idx = jnp.concatenate([E3[:, TSTART:], pad3], axis=1)
    tail_w = jnp.concatenate(
        [Ew2[TSTART:], jnp.asarray(np.zeros((npadg, G), np.float32))], axis=0)

    b1r = b1.reshape(1, D)
    b3r = b3.reshape(1, D)

    a = _lin1(x, W1, b1r)
    S, deg = _edge_scatter(a, E3, Ew2, tail_idx, tail_w)
    d0 = deg[0, :N].reshape(N, 1)
    d1 = deg[1, :N].reshape(N, 1)
    return _combine(x, S, d0, d1, W2, W3, b3r)
